# baseline (speedup 1.0000x reference)
import functools

import jax
import jax.numpy as jnp
from jax import lax
from jax.experimental import pallas as pl
from jax.experimental.pallas import tpu as pltpu
from jax.experimental.pallas import tpu_sc as plsc

B, S, D, H, FF, E = 2, 2048, 1024, 16, 4096, 8
DH = D // H
N = B * S
TM = 256
NPAD = N + E * TM
NT = NPAD // TM
QB = 512
RT = 512

_SQRT_HALF = 0.7071067811865476


def _qkv_body(x_ref, w_ref, b_ref, o_ref):
    acc = lax.dot_general(x_ref[...], w_ref[...], (((1,), (1,)), ((), ())),
                          preferred_element_type=jnp.float32,
                          precision=lax.Precision.HIGHEST)
    acc = acc + b_ref[0, 0, :][None, :]
    for i in range(4):
        o_ref[i] = acc[:, i * DH:(i + 1) * DH]


def _attn_body(q_ref, k_ref, v_ref, o_ref):
    s = lax.dot_general(q_ref[0], k_ref[0], (((1,), (1,)), ((), ())),
                        preferred_element_type=jnp.float32,
                        precision=lax.Precision.HIGHEST)
    s = s * (1.0 / 8.0)
    m = jnp.max(s, axis=1, keepdims=True)
    p = jnp.exp(s - m)
    l = jnp.sum(p, axis=1, keepdims=True)
    p = p / l
    o_ref[0] = lax.dot_general(p, v_ref[0], (((1,), (0,)), ((), ())),
                               preferred_element_type=jnp.float32,
                               precision=lax.Precision.HIGHEST)


def _post_attn_body(a_ref, wo_ref, bo_ref, x_ref, g_ref, b_ref, gw_ref,
                    x1_ref, idx_ref):
    o = lax.dot_general(a_ref[...], wo_ref[...], (((1,), (1,)), ((), ())),
                        preferred_element_type=jnp.float32,
                        precision=lax.Precision.HIGHEST)
    o = o + bo_ref[0, :][None, :] + x_ref[...]
    mu = jnp.mean(o, axis=1, keepdims=True)
    var = jnp.mean((o - mu) ** 2, axis=1, keepdims=True)
    x1 = (o - mu) * lax.rsqrt(var + 1e-5)
    x1 = x1 * g_ref[0, :][None, :] + b_ref[0, :][None, :]
    x1_ref[...] = x1
    logits = lax.dot_general(x1, gw_ref[...], (((1,), (0,)), ((), ())),
                             preferred_element_type=jnp.float32)
    col = lax.broadcasted_iota(jnp.int32, logits.shape, 1)
    logits = jnp.where(col < E, logits, -1e30)
    mx = jnp.max(logits, axis=1, keepdims=True)
    cand = jnp.where(logits >= mx, col, 128)
    idx_ref[...] = jnp.min(cand, axis=1).astype(jnp.int32)


def _moe_body(te_ref, xs_ref, w1_ref, b1_ref, w2_ref, b2_ref, o_ref):
    xb = xs_ref[...].astype(jnp.bfloat16)
    h = lax.dot_general(xb, w1_ref[0], (((1,), (0,)), ((), ())),
                        preferred_element_type=jnp.float32)
    h = h + b1_ref[0, 0, :][None, :]
    h = 0.5 * h * (1.0 + lax.erf(h * _SQRT_HALF))
    o = lax.dot_general(h.astype(jnp.bfloat16), w2_ref[0],
                        (((1,), (0,)), ((), ())),
                        preferred_element_type=jnp.float32)
    o_ref[...] = o + b2_ref[0, 0, :][None, :]


def _ln2_body(x1_ref, mo_ref, g_ref, b_ref, o_ref):
    o = x1_ref[...] + mo_ref[...]
    mu = jnp.mean(o, axis=1, keepdims=True)
    var = jnp.mean((o - mu) ** 2, axis=1, keepdims=True)
    o = (o - mu) * lax.rsqrt(var + 1e-5)
    o_ref[...] = o * g_ref[0, :][None, :] + b_ref[0, :][None, :]


def _sc_gather_rows(table, idx, n_out):
    NC, NS = 2, 16
    NW = NC * NS
    per_w = n_out // NW
    CH = 64
    n_ch = per_w // CH
    mesh = plsc.VectorSubcoreMesh(core_axis_name="c", subcore_axis_name="s")

    @functools.partial(
        pl.kernel, mesh=mesh,
        out_type=jax.ShapeDtypeStruct((n_out, D), jnp.float32),
        scratch_types=[
            pltpu.VMEM((CH,), jnp.int32),
            pltpu.VMEM((CH, D), jnp.float32),
            pltpu.SemaphoreType.DMA,
        ],
    )
    def k(table_hbm, idx_hbm, out_hbm, idx_v, rows_v, sem):
        wid = lax.axis_index("s") * NC + lax.axis_index("c")
        base = wid * per_w

        def body(c, carry):
            off = base + c * CH
            pltpu.sync_copy(idx_hbm.at[pl.ds(off, CH)], idx_v)
            pltpu.async_copy(table_hbm.at[idx_v], rows_v, sem).wait()
            pltpu.sync_copy(rows_v, out_hbm.at[pl.ds(off, CH)])
            return carry

        lax.fori_loop(0, n_ch, body, 0)

    return k(table, idx)


def kernel(x, in_proj_w, in_proj_b, out_proj_w, out_proj_b, ln1_g, ln1_b,
           ln2_g, ln2_b, gate_w, w1, b1, w2, b2):
    xf = x.reshape(N, D)
    xb = xf
    wqkv = in_proj_w
    bqkv3 = in_proj_b.reshape(12, 1, 256)
    wo = out_proj_w
    gwp = jnp.pad(gate_w, ((0, 0), (0, 128 - E)))
    w1b = w1.astype(jnp.bfloat16)
    w2b = w2.astype(jnp.bfloat16)
    b13 = b1.reshape(E, 1, FF)
    b23 = b2.reshape(E, 1, D)

    qkv = pl.pallas_call(
        _qkv_body,
        grid=(12, N // RT),
        in_specs=[
            pl.BlockSpec((RT, D), lambda c, r: (r, 0)),
            pl.BlockSpec((256, D), lambda c, r: (c, 0)),
            pl.BlockSpec((1, 1, 256), lambda c, r: (c, 0, 0)),
        ],
        out_specs=pl.BlockSpec((4, RT, DH), lambda c, r: (c, r, 0)),
        out_shape=jax.ShapeDtypeStruct((3 * H, N, DH), jnp.float32),
    )(xb, wqkv, bqkv3)

    attn3 = pl.pallas_call(
        _attn_body,
        grid=(B, H, S // QB),
        in_specs=[
            pl.BlockSpec((1, QB, DH), lambda b, h, q: (h, b * (S // QB) + q, 0)),
            pl.BlockSpec((1, S, DH), lambda b, h, q: (H + h, b, 0)),
            pl.BlockSpec((1, S, DH), lambda b, h, q: (2 * H + h, b, 0)),
        ],
        out_specs=pl.BlockSpec((1, QB, DH),
                               lambda b, h, q: (h, b * (S // QB) + q, 0)),
        out_shape=jax.ShapeDtypeStruct((H, N, DH), jnp.float32),
    )(qkv, qkv, qkv)
    attn = attn3.transpose(1, 0, 2).reshape(N, D)

    x1, idx = pl.pallas_call(
        _post_attn_body,
        grid=(N // RT,),
        in_specs=[
            pl.BlockSpec((RT, D), lambda r: (r, 0)),
            pl.BlockSpec((D, D), lambda r: (0, 0)),
            pl.BlockSpec((1, D), lambda r: (0, 0)),
            pl.BlockSpec((RT, D), lambda r: (r, 0)),
            pl.BlockSpec((1, D), lambda r: (0, 0)),
            pl.BlockSpec((1, D), lambda r: (0, 0)),
            pl.BlockSpec((D, 128), lambda r: (0, 0)),
        ],
        out_specs=[
            pl.BlockSpec((RT, D), lambda r: (r, 0)),
            pl.BlockSpec((RT,), lambda r: (r,)),
        ],
        out_shape=[
            jax.ShapeDtypeStruct((N, D), jnp.float32),
            jax.ShapeDtypeStruct((N,), jnp.int32),
        ],
    )(attn, wo, out_proj_b.reshape(1, D), xf, ln1_g.reshape(1, D),
      ln1_b.reshape(1, D), gwp)

    onehot = (idx[:, None] == jnp.arange(E, dtype=jnp.int32)[None, :]
              ).astype(jnp.int32)
    counts = jnp.sum(onehot, axis=0)
    rank = jnp.take_along_axis(jnp.cumsum(onehot, axis=0), idx[:, None],
                               axis=1)[:, 0] - 1
    tile_counts = (counts + TM - 1) // TM
    tile_offs = jnp.concatenate([jnp.zeros((1,), jnp.int32),
                                 jnp.cumsum(tile_counts)[:-1].astype(jnp.int32)])
    offs = tile_offs * TM
    dest = offs[idx] + rank
    src = jnp.zeros((NPAD,), jnp.int32).at[dest].set(
        jnp.arange(N, dtype=jnp.int32))
    te = jnp.sum((jnp.arange(NT, dtype=jnp.int32)[:, None]
                  >= tile_offs[None, 1:]).astype(jnp.int32), axis=1)

    xs = _sc_gather_rows(x1, src, NPAD)

    grid_spec = pltpu.PrefetchScalarGridSpec(
        num_scalar_prefetch=1,
        grid=(NT,),
        in_specs=[
            pl.BlockSpec((TM, D), lambda i, te: (i, 0)),
            pl.BlockSpec((1, D, FF), lambda i, te: (te[i], 0, 0)),
            pl.BlockSpec((1, 1, FF), lambda i, te: (te[i], 0, 0)),
            pl.BlockSpec((1, FF, D), lambda i, te: (te[i], 0, 0)),
            pl.BlockSpec((1, 1, D), lambda i, te: (te[i], 0, 0)),
        ],
        out_specs=pl.BlockSpec((TM, D), lambda i, te: (i, 0)),
    )
    mo_pad = pl.pallas_call(
        _moe_body,
        grid_spec=grid_spec,
        out_shape=jax.ShapeDtypeStruct((NPAD, D), jnp.float32),
    )(te, xs, w1b, b13, w2b, b23)

    mo = _sc_gather_rows(mo_pad, dest, N)

    out = pl.pallas_call(
        _ln2_body,
        grid=(N // RT,),
        in_specs=[
            pl.BlockSpec((RT, D), lambda r: (r, 0)),
            pl.BlockSpec((RT, D), lambda r: (r, 0)),
            pl.BlockSpec((1, D), lambda r: (0, 0)),
            pl.BlockSpec((1, D), lambda r: (0, 0)),
        ],
        out_specs=pl.BlockSpec((RT, D), lambda r: (r, 0)),
        out_shape=jax.ShapeDtypeStruct((N, D), jnp.float32),
    )(x1, mo, ln2_g.reshape(1, D), ln2_b.reshape(1, D))

    return out.reshape(B, S, D)


# speedup vs baseline: 1.5305x; 1.5305x over previous
import functools

import jax
import jax.numpy as jnp
from jax import lax
from jax.experimental import pallas as pl
from jax.experimental.pallas import tpu as pltpu
from jax.experimental.pallas import tpu_sc as plsc

B, S, D, H, FF, E = 2, 2048, 1024, 16, 4096, 8
DH = D // H
N = B * S
TM = 256
NPAD = N + E * TM
NT = NPAD // TM
QB = 512
RT = 512

_SQRT_HALF = 0.7071067811865476


def _dot3(a, b, dims):
    ah = a.astype(jnp.bfloat16)
    al = (a - ah.astype(jnp.float32)).astype(jnp.bfloat16)
    bh = b.astype(jnp.bfloat16)
    bl = (b - bh.astype(jnp.float32)).astype(jnp.bfloat16)
    d = lambda u, v: lax.dot_general(u, v, dims,
                                     preferred_element_type=jnp.float32)
    return d(ah, bh) + d(ah, bl) + d(al, bh)


def _qkv_body(x_ref, w_ref, b_ref, o_ref):
    acc = _dot3(x_ref[...], w_ref[...], (((1,), (1,)), ((), ())))
    acc = acc + b_ref[0, 0, :][None, :]
    for i in range(4):
        o_ref[i] = acc[:, i * DH:(i + 1) * DH]


def _attn_body(q_ref, k_ref, v_ref, o_ref):
    s = _dot3(q_ref[0], k_ref[0], (((1,), (1,)), ((), ())))
    s = s * (1.0 / 8.0)
    m = jnp.max(s, axis=1, keepdims=True)
    p = jnp.exp(s - m)
    l = jnp.sum(p, axis=1, keepdims=True)
    p = p / l
    o_ref[0] = _dot3(p, v_ref[0], (((1,), (0,)), ((), ())))


def _post_attn_body(a_ref, wo_ref, bo_ref, x_ref, g_ref, b_ref, gw_ref,
                    x1_ref, idx_ref):
    o = _dot3(a_ref[...], wo_ref[...], (((1,), (1,)), ((), ())))
    o = o + bo_ref[0, :][None, :] + x_ref[...]
    mu = jnp.mean(o, axis=1, keepdims=True)
    var = jnp.mean((o - mu) ** 2, axis=1, keepdims=True)
    x1 = (o - mu) * lax.rsqrt(var + 1e-5)
    x1 = x1 * g_ref[0, :][None, :] + b_ref[0, :][None, :]
    x1_ref[...] = x1
    logits = lax.dot_general(x1, gw_ref[...], (((1,), (0,)), ((), ())),
                             preferred_element_type=jnp.float32)
    col = lax.broadcasted_iota(jnp.int32, logits.shape, 1)
    logits = jnp.where(col < E, logits, -1e30)
    mx = jnp.max(logits, axis=1, keepdims=True)
    cand = jnp.where(logits >= mx, col, 128)
    idx_ref[...] = jnp.min(cand, axis=1).astype(jnp.int32)


def _moe_body(te_ref, xs_ref, w1_ref, b1_ref, w2_ref, b2_ref, o_ref):
    xb = xs_ref[...].astype(jnp.bfloat16)
    h = lax.dot_general(xb, w1_ref[0], (((1,), (0,)), ((), ())),
                        preferred_element_type=jnp.float32)
    h = h + b1_ref[0, 0, :][None, :]
    h = 0.5 * h * (1.0 + lax.erf(h * _SQRT_HALF))
    o = lax.dot_general(h.astype(jnp.bfloat16), w2_ref[0],
                        (((1,), (0,)), ((), ())),
                        preferred_element_type=jnp.float32)
    o_ref[...] = o + b2_ref[0, 0, :][None, :]


def _ln2_body(x1_ref, mo_ref, g_ref, b_ref, o_ref):
    o = x1_ref[...] + mo_ref[...]
    mu = jnp.mean(o, axis=1, keepdims=True)
    var = jnp.mean((o - mu) ** 2, axis=1, keepdims=True)
    o = (o - mu) * lax.rsqrt(var + 1e-5)
    o_ref[...] = o * g_ref[0, :][None, :] + b_ref[0, :][None, :]


def _sc_gather_rows(table, idx, n_out):
    NC, NS = 2, 16
    NW = NC * NS
    per_w = n_out // NW
    CH = 64
    n_ch = per_w // CH
    mesh = plsc.VectorSubcoreMesh(core_axis_name="c", subcore_axis_name="s")

    @functools.partial(
        pl.kernel, mesh=mesh,
        out_type=jax.ShapeDtypeStruct((n_out, D), jnp.float32),
        scratch_types=[
            pltpu.VMEM((CH,), jnp.int32),
            pltpu.VMEM((CH, D), jnp.float32),
            pltpu.SemaphoreType.DMA,
        ],
    )
    def k(table_hbm, idx_hbm, out_hbm, idx_v, rows_v, sem):
        wid = lax.axis_index("s") * NC + lax.axis_index("c")
        base = wid * per_w

        def body(c, carry):
            off = base + c * CH
            pltpu.sync_copy(idx_hbm.at[pl.ds(off, CH)], idx_v)
            pltpu.async_copy(table_hbm.at[idx_v], rows_v, sem).wait()
            pltpu.sync_copy(rows_v, out_hbm.at[pl.ds(off, CH)])
            return carry

        lax.fori_loop(0, n_ch, body, 0)

    return k(table, idx)


def kernel(x, in_proj_w, in_proj_b, out_proj_w, out_proj_b, ln1_g, ln1_b,
           ln2_g, ln2_b, gate_w, w1, b1, w2, b2):
    xf = x.reshape(N, D)
    xb = xf
    wqkv = in_proj_w
    bqkv3 = in_proj_b.reshape(12, 1, 256)
    wo = out_proj_w
    gwp = jnp.pad(gate_w, ((0, 0), (0, 128 - E)))
    w1b = w1.astype(jnp.bfloat16)
    w2b = w2.astype(jnp.bfloat16)
    b13 = b1.reshape(E, 1, FF)
    b23 = b2.reshape(E, 1, D)

    qkv = pl.pallas_call(
        _qkv_body,
        grid=(12, N // RT),
        in_specs=[
            pl.BlockSpec((RT, D), lambda c, r: (r, 0)),
            pl.BlockSpec((256, D), lambda c, r: (c, 0)),
            pl.BlockSpec((1, 1, 256), lambda c, r: (c, 0, 0)),
        ],
        out_specs=pl.BlockSpec((4, RT, DH), lambda c, r: (c, r, 0)),
        out_shape=jax.ShapeDtypeStruct((3 * H, N, DH), jnp.float32),
    )(xb, wqkv, bqkv3)

    attn3 = pl.pallas_call(
        _attn_body,
        grid=(B, H, S // QB),
        in_specs=[
            pl.BlockSpec((1, QB, DH), lambda b, h, q: (h, b * (S // QB) + q, 0)),
            pl.BlockSpec((1, S, DH), lambda b, h, q: (H + h, b, 0)),
            pl.BlockSpec((1, S, DH), lambda b, h, q: (2 * H + h, b, 0)),
        ],
        out_specs=pl.BlockSpec((1, QB, DH),
                               lambda b, h, q: (h, b * (S // QB) + q, 0)),
        out_shape=jax.ShapeDtypeStruct((H, N, DH), jnp.float32),
    )(qkv, qkv, qkv)
    attn = attn3.transpose(1, 0, 2).reshape(N, D)

    x1, idx = pl.pallas_call(
        _post_attn_body,
        grid=(N // RT,),
        in_specs=[
            pl.BlockSpec((RT, D), lambda r: (r, 0)),
            pl.BlockSpec((D, D), lambda r: (0, 0)),
            pl.BlockSpec((1, D), lambda r: (0, 0)),
            pl.BlockSpec((RT, D), lambda r: (r, 0)),
            pl.BlockSpec((1, D), lambda r: (0, 0)),
            pl.BlockSpec((1, D), lambda r: (0, 0)),
            pl.BlockSpec((D, 128), lambda r: (0, 0)),
        ],
        out_specs=[
            pl.BlockSpec((RT, D), lambda r: (r, 0)),
            pl.BlockSpec((RT,), lambda r: (r,)),
        ],
        out_shape=[
            jax.ShapeDtypeStruct((N, D), jnp.float32),
            jax.ShapeDtypeStruct((N,), jnp.int32),
        ],
    )(attn, wo, out_proj_b.reshape(1, D), xf, ln1_g.reshape(1, D),
      ln1_b.reshape(1, D), gwp)

    onehot = (idx[:, None] == jnp.arange(E, dtype=jnp.int32)[None, :]
              ).astype(jnp.int32)
    counts = jnp.sum(onehot, axis=0)
    rank = jnp.take_along_axis(jnp.cumsum(onehot, axis=0), idx[:, None],
                               axis=1)[:, 0] - 1
    tile_counts = (counts + TM - 1) // TM
    tile_offs = jnp.concatenate([jnp.zeros((1,), jnp.int32),
                                 jnp.cumsum(tile_counts)[:-1].astype(jnp.int32)])
    offs = tile_offs * TM
    dest = offs[idx] + rank
    src = jnp.zeros((NPAD,), jnp.int32).at[dest].set(
        jnp.arange(N, dtype=jnp.int32))
    te = jnp.sum((jnp.arange(NT, dtype=jnp.int32)[:, None]
                  >= tile_offs[None, 1:]).astype(jnp.int32), axis=1)

    xs = _sc_gather_rows(x1, src, NPAD)

    grid_spec = pltpu.PrefetchScalarGridSpec(
        num_scalar_prefetch=1,
        grid=(NT,),
        in_specs=[
            pl.BlockSpec((TM, D), lambda i, te: (i, 0)),
            pl.BlockSpec((1, D, FF), lambda i, te: (te[i], 0, 0)),
            pl.BlockSpec((1, 1, FF), lambda i, te: (te[i], 0, 0)),
            pl.BlockSpec((1, FF, D), lambda i, te: (te[i], 0, 0)),
            pl.BlockSpec((1, 1, D), lambda i, te: (te[i], 0, 0)),
        ],
        out_specs=pl.BlockSpec((TM, D), lambda i, te: (i, 0)),
    )
    mo_pad = pl.pallas_call(
        _moe_body,
        grid_spec=grid_spec,
        out_shape=jax.ShapeDtypeStruct((NPAD, D), jnp.float32),
    )(te, xs, w1b, b13, w2b, b23)

    mo = _sc_gather_rows(mo_pad, dest, N)

    out = pl.pallas_call(
        _ln2_body,
        grid=(N // RT,),
        in_specs=[
            pl.BlockSpec((RT, D), lambda r: (r, 0)),
            pl.BlockSpec((RT, D), lambda r: (r, 0)),
            pl.BlockSpec((1, D), lambda r: (0, 0)),
            pl.BlockSpec((1, D), lambda r: (0, 0)),
        ],
        out_specs=pl.BlockSpec((RT, D), lambda r: (r, 0)),
        out_shape=jax.ShapeDtypeStruct((N, D), jnp.float32),
    )(x1, mo, ln2_g.reshape(1, D), ln2_b.reshape(1, D))

    return out.reshape(B, S, D)
